# baseline (speedup 1.0000x reference)
import jax
import jax.numpy as jnp
from jax import lax
from jax.experimental import pallas as pl
from jax.experimental.pallas import tpu as pltpu
from jax.experimental.pallas import tpu_sc as plsc

_B, _N, _C = 16, 1024, 1024
_ROWS = _B * _N
_NW = 32
_G = 16
_PW = _ROWS // _NW
_NBATCH = _PW // _G
_NCH = 8 * _G

_mesh = plsc.VectorSubcoreMesh(core_axis_name="c", subcore_axis_name="s")


def _sc_body(x_hbm, o_hbm, idx_in, idx_out, data, outv, sem_in, sem_out):
    wid = lax.axis_index("s") * 2 + lax.axis_index("c")
    lane = jax.lax.iota(jnp.int32, 16)
    half = jnp.full((16,), 0.5, dtype=jnp.float32)
    zero = jnp.zeros((16,), dtype=jnp.float32)

    def batch(t):
        r0 = wid * _PW + t * _G
        for q in range(_NCH // 16):
            p = q * 16 + lane
            r = r0 + (p >> 3)
            idx_in[pl.ds(q * 16, 16)] = (r << 4) + ((p & 7) << 1) + 1
            rr = r & (_N - 1)
            idx_out[pl.ds(q * 16, 16)] = (
                ((r >> 10) << 13) + ((rr >> 3) << 6) + ((p & 7) << 3) + (r & 7)
            )
        cp_in = pltpu.make_async_copy(x_hbm.at[idx_in], data, sem_in)
        cp_in.start()
        cp_in.wait()

        def row(g):
            rf = jnp.full((16,), 0.0, dtype=jnp.float32) + (
                (r0 + g) & (_N - 1)
            ).astype(jnp.float32)
            for j in range(8):
                for k in range(8):
                    v = data[g * 8 + j, pl.ds(k * 16, 16)]
                    outv[g * 8 + j, pl.ds(k * 16, 16)] = jnp.where(
                        v > half, rf, zero
                    )

        pl.loop(0, _G)(row)
        cp_out = pltpu.make_async_copy(outv, o_hbm.at[idx_out], sem_out)
        cp_out.start()
        cp_out.wait()

    pl.loop(0, _NBATCH)(batch)


def kernel(x):
    xin = jnp.transpose(
        x.reshape(_ROWS, _C // 128, 128, 2), (0, 1, 3, 2)
    ).reshape(_ROWS * 16, 128)
    sck = pl.kernel(
        _sc_body,
        out_type=jax.ShapeDtypeStruct((_ROWS * 8, 128), jnp.float32),
        mesh=_mesh,
        scratch_types=[
            pltpu.VMEM((_NCH,), jnp.int32),
            pltpu.VMEM((_NCH,), jnp.int32),
            pltpu.VMEM((_NCH, 128), jnp.float32),
            pltpu.VMEM((_NCH, 128), jnp.float32),
            pltpu.SemaphoreType.DMA,
            pltpu.SemaphoreType.DMA,
        ],
    )
    out = sck(xin)
    return (
        out.reshape(_B, _N // 8, 8, 8, 128)
        .transpose(0, 1, 3, 2, 4)
        .reshape(_B, _N, _C)
    )


# speedup vs baseline: 1.5502x; 1.5502x over previous
import jax
import jax.numpy as jnp
from jax import lax
from jax.experimental import pallas as pl
from jax.experimental.pallas import tpu as pltpu
from jax.experimental.pallas import tpu_sc as plsc

_B, _N, _C = 16, 1024, 1024
_ROWS = _B * _N
_NW = 32
_G = 16
_PW = _ROWS // _NW
_NB = _PW // _G
_NCH = 8 * _G

_mesh = plsc.VectorSubcoreMesh(core_axis_name="c", subcore_axis_name="s")


def _sc_body(
    x_hbm,
    o_hbm,
    idx_in0,
    idx_in1,
    idx_out0,
    idx_out1,
    data0,
    data1,
    outv0,
    outv1,
    sg0,
    sg1,
    ss0,
    ss1,
):
    idx_in = (idx_in0, idx_in1)
    idx_out = (idx_out0, idx_out1)
    data = (data0, data1)
    outv = (outv0, outv1)
    sg = (sg0, sg1)
    ss = (ss0, ss1)

    wid = lax.axis_index("s") * 2 + lax.axis_index("c")
    r0w = wid * _PW
    lane = jax.lax.iota(jnp.int32, 16)
    half = jnp.full((16,), 0.5, dtype=jnp.float32)
    zero = jnp.zeros((16,), dtype=jnp.float32)

    for b in range(2):
        r00 = r0w + b * _G
        for q in range(_NCH // 16):
            p = q * 16 + lane
            r = r00 + (p >> 3)
            idx_in[b][pl.ds(q * 16, 16)] = (r << 4) + ((p & 7) << 1) + 1
            rr = r & (_N - 1)
            idx_out[b][pl.ds(q * 16, 16)] = (
                ((r >> 10) << 13) + ((rr >> 3) << 6) + ((p & 7) << 3) + (r & 7)
            )

    def gather(b):
        return pltpu.make_async_copy(x_hbm.at[idx_in[b]], data[b], sg[b])

    def scatter(b):
        return pltpu.make_async_copy(outv[b], o_hbm.at[idx_out[b]], ss[b])

    gather(0).start()
    gather(1).start()

    def super_batch(t):
        for b in range(2):
            m = t + b
            gather(b).wait()

            @pl.when(m >= 2)
            def _():
                scatter(b).wait()
                for q in range(_NCH // 16):
                    sl = pl.ds(q * 16, 16)
                    idx_out[b][sl] = idx_out[b][sl] + (2 * _G * 8)

            def row(g):
                rf = jnp.full((16,), 0.0, dtype=jnp.float32) + (
                    (r0w + m * _G + g) & (_N - 1)
                ).astype(jnp.float32)
                for j in range(8):
                    for k in range(8):
                        v = data[b][g * 8 + j, pl.ds(k * 16, 16)]
                        outv[b][g * 8 + j, pl.ds(k * 16, 16)] = jnp.where(
                            v > half, rf, zero
                        )

            pl.loop(0, _G)(row)
            scatter(b).start()

            @pl.when(m + 2 <= _NB - 1)
            def _():
                for q in range(_NCH // 16):
                    sl = pl.ds(q * 16, 16)
                    idx_in[b][sl] = idx_in[b][sl] + (2 * _G * 16)
                gather(b).start()

    pl.loop(0, _NB, step=2)(super_batch)
    scatter(0).wait()
    scatter(1).wait()


def kernel(x):
    xin = jnp.transpose(
        x.reshape(_ROWS, _C // 128, 128, 2), (0, 1, 3, 2)
    ).reshape(_ROWS * 16, 128)
    sck = pl.kernel(
        _sc_body,
        out_type=jax.ShapeDtypeStruct((_ROWS * 8, 128), jnp.float32),
        mesh=_mesh,
        scratch_types=[
            pltpu.VMEM((_NCH,), jnp.int32),
            pltpu.VMEM((_NCH,), jnp.int32),
            pltpu.VMEM((_NCH,), jnp.int32),
            pltpu.VMEM((_NCH,), jnp.int32),
            pltpu.VMEM((_NCH, 128), jnp.float32),
            pltpu.VMEM((_NCH, 128), jnp.float32),
            pltpu.VMEM((_NCH, 128), jnp.float32),
            pltpu.VMEM((_NCH, 128), jnp.float32),
            pltpu.SemaphoreType.DMA,
            pltpu.SemaphoreType.DMA,
            pltpu.SemaphoreType.DMA,
            pltpu.SemaphoreType.DMA,
        ],
    )
    out = sck(xin)
    return (
        out.reshape(_B, _N // 8, 8, 8, 128)
        .transpose(0, 1, 3, 2, 4)
        .reshape(_B, _N, _C)
    )


# speedup vs baseline: 1.5907x; 1.0261x over previous
import jax
import jax.numpy as jnp
from jax import lax
from jax.experimental import pallas as pl
from jax.experimental.pallas import tpu as pltpu
from jax.experimental.pallas import tpu_sc as plsc

_B, _N, _C = 16, 1024, 1024
_ROWS = _B * _N
_NW = 32
_G = 8
_PW = _ROWS // _NW
_NB = _PW // _G
_NCH = 8 * _G
_D = 4

_mesh = plsc.VectorSubcoreMesh(core_axis_name="c", subcore_axis_name="s")


def _sc_body(x_hbm, o_hbm, *refs):
    idx_in = refs[0:_D]
    idx_out = refs[_D : 2 * _D]
    data = refs[2 * _D : 3 * _D]
    outv = refs[3 * _D : 4 * _D]
    sg = refs[4 * _D : 5 * _D]
    ss = refs[5 * _D : 6 * _D]

    wid = lax.axis_index("s") * 2 + lax.axis_index("c")
    r0w = wid * _PW
    lane = jax.lax.iota(jnp.int32, 16)
    half = jnp.full((16,), 0.5, dtype=jnp.float32)
    zero = jnp.zeros((16,), dtype=jnp.float32)

    for b in range(_D):
        r00 = r0w + b * _G
        for q in range(_NCH // 16):
            p = q * 16 + lane
            r = r00 + (p >> 3)
            idx_in[b][pl.ds(q * 16, 16)] = (r << 4) + ((p & 7) << 1) + 1
            rr = r & (_N - 1)
            idx_out[b][pl.ds(q * 16, 16)] = (
                ((r >> 10) << 13) + ((rr >> 3) << 6) + ((p & 7) << 3) + (r & 7)
            )

    def gather(b):
        return pltpu.make_async_copy(x_hbm.at[idx_in[b]], data[b], sg[b])

    def scatter(b):
        return pltpu.make_async_copy(outv[b], o_hbm.at[idx_out[b]], ss[b])

    for b in range(_D):
        gather(b).start()

    def super_batch(t):
        for b in range(_D):
            m = t + b
            gather(b).wait()

            @pl.when(m >= _D)
            def _():
                scatter(b).wait()
                for q in range(_NCH // 16):
                    sl = pl.ds(q * 16, 16)
                    idx_out[b][sl] = idx_out[b][sl] + (_D * _G * 8)

            def row(g):
                rf = jnp.full((16,), 0.0, dtype=jnp.float32) + (
                    (r0w + m * _G + g) & (_N - 1)
                ).astype(jnp.float32)
                for j in range(8):
                    for k in range(8):
                        v = data[b][g * 8 + j, pl.ds(k * 16, 16)]
                        outv[b][g * 8 + j, pl.ds(k * 16, 16)] = jnp.where(
                            v > half, rf, zero
                        )

            pl.loop(0, _G)(row)
            scatter(b).start()

            @pl.when(m + _D <= _NB - 1)
            def _():
                for q in range(_NCH // 16):
                    sl = pl.ds(q * 16, 16)
                    idx_in[b][sl] = idx_in[b][sl] + (_D * _G * 16)
                gather(b).start()

    pl.loop(0, _NB, step=_D)(super_batch)
    for b in range(_D):
        scatter(b).wait()


def kernel(x):
    xin = jnp.transpose(
        x.reshape(_ROWS, _C // 128, 128, 2), (0, 1, 3, 2)
    ).reshape(_ROWS * 16, 128)
    sck = pl.kernel(
        _sc_body,
        out_type=jax.ShapeDtypeStruct((_ROWS * 8, 128), jnp.float32),
        mesh=_mesh,
        scratch_types=(
            [pltpu.VMEM((_NCH,), jnp.int32) for _ in range(2 * _D)]
            + [pltpu.VMEM((_NCH, 128), jnp.float32) for _ in range(2 * _D)]
            + [pltpu.SemaphoreType.DMA for _ in range(2 * _D)]
        ),
    )
    out = sck(xin)
    return (
        out.reshape(_B, _N // 8, 8, 8, 128)
        .transpose(0, 1, 3, 2, 4)
        .reshape(_B, _N, _C)
    )
